# baseline (speedup 1.0000x reference)
import functools

import jax
import jax.numpy as jnp
from jax.experimental import pallas as pl
from jax.experimental.pallas import tpu as pltpu


def _se_body(x_ref, ones_ref, w1t_ref, w2t_ref, o_ref, *, inv_hw):
    x = x_ref[0]
    psum = jax.lax.dot_general(
        x, ones_ref[...], (((1,), (0,)), ((), ())),
        preferred_element_type=jnp.float32)
    pooled = psum * inv_hw
    hidden = jnp.maximum(
        jax.lax.dot_general(w1t_ref[...], pooled,
                            (((1,), (0,)), ((), ())),
                            preferred_element_type=jnp.float32), 0.0)
    gate = jax.nn.sigmoid(
        jax.lax.dot_general(w2t_ref[...], hidden,
                            (((1,), (0,)), ((), ())),
                            preferred_element_type=jnp.float32))
    o_ref[0] = x * gate[:, :1]


def kernel(x_nchw, w1, w2):
    B, C, H, W = x_nchw.shape
    Cr = w1.shape[1]
    HW = H * W
    x_flat = x_nchw.reshape(B, C, HW)

    out_flat = pl.pallas_call(
        functools.partial(_se_body, inv_hw=1.0 / float(HW)),
        out_shape=jax.ShapeDtypeStruct((B, C, HW), x_nchw.dtype),
        grid=(B,),
        in_specs=[
            pl.BlockSpec((1, C, HW), lambda b: (b, 0, 0)),
            pl.BlockSpec((HW, 128), lambda b: (0, 0)),
            pl.BlockSpec((Cr, C), lambda b: (0, 0)),
            pl.BlockSpec((C, Cr), lambda b: (0, 0)),
        ],
        out_specs=pl.BlockSpec((1, C, HW), lambda b: (b, 0, 0)),
        compiler_params=pltpu.CompilerParams(
            dimension_semantics=("parallel",),
            vmem_limit_bytes=40 * 1024 * 1024),
    )(x_flat, jnp.ones((HW, 128), jnp.float32), w1.T, w2.T)
    return out_flat.reshape(B, C, H, W)
